# baseline (speedup 1.0000x reference)
import jax
import jax.numpy as jnp
from jax.experimental import pallas as pl
from jax.experimental.pallas import tpu as pltpu

_UNROLL = 64
_TREE = 32


def _vmse_kernel(nv_ref, p_ref, g_ref, w_ref, o_ref, a_ref, b_ref):
    n_pts = a_ref.shape[0]
    bi = a_ref.shape[2]
    b_tot = b_ref.shape[2]

    pt3 = p_ref[:].T.reshape(n_pts, 3, b_tot)
    a_ref[:, 0:3, :] = pt3
    a_ref[:, 3:4, :] = jnp.sum(pt3 * pt3, axis=1, keepdims=True)
    a_ref[:, 4:5, :] = jnp.ones((n_pts, 1, b_tot), jnp.float32)
    a_ref[:, 5:8, :] = jnp.zeros((n_pts, 3, b_tot), jnp.float32)
    gt3 = g_ref[:].T.reshape(n_pts, 3, b_tot)
    b_ref[:, 0:3, :] = -2.0 * gt3
    b_ref[:, 3:4, :] = jnp.ones((n_pts, 1, b_tot), jnp.float32)
    b_ref[:, 4:5, :] = jnp.sum(gt3 * gt3, axis=1, keepdims=True)
    b_ref[:, 5:8, :] = jnp.zeros((n_pts, 3, b_tot), jnp.float32)

    def dist(n):
        a = a_ref[pl.ds(n, 1)].reshape(8, bi)
        bb = b_ref[pl.ds(n, 1)].reshape(8, b_tot)
        d2 = jax.lax.dot_general(a, bb, (((0,), (0,)), ((), ())),
                                 preferred_element_type=jnp.float32)
        mb = jnp.maximum(d2.astype(jnp.bfloat16),
                         jnp.bfloat16(1e-30))
        return mb * jax.lax.rsqrt(mb)

    parts = []
    for t in range(n_pts // _TREE):
        s = dist(t * _TREE)
        for u in range(1, _TREE):
            s = s + dist(t * _TREE + u)
        parts.append(s.astype(jnp.float32))
    while len(parts) > 1:
        parts = [parts[i] + parts[i + 1] for i in range(0, len(parts), 2)]
    acc = parts[0]

    nv = nv_ref[0, 0]
    norms = acc * (1.0 / n_pts)
    logits = (norms * norms) * (-0.5 / nv)
    m = jnp.max(logits, axis=1, keepdims=True)
    ex = jnp.exp(logits - m)
    lse = jnp.log(jnp.sum(ex, axis=1, keepdims=True)) + m
    rows = jax.lax.broadcasted_iota(jnp.int32, (bi, b_tot), 0)
    cols = jax.lax.broadcasted_iota(jnp.int32, (bi, b_tot), 1)
    diag = jnp.sum(jnp.where(rows == cols, logits, 0.0), axis=1,
                   keepdims=True)
    loss = (lse - diag) * (2.0 * nv) * w_ref[:, :1]
    o_ref[:, :] = jnp.broadcast_to(loss, (bi, 128))


@jax.jit
def kernel(pred, gt, weights, sigma):
    B, N, D = pred.shape
    f32 = jnp.float32
    pred = pred.astype(f32)
    gt = gt.astype(f32)

    pred_r = pred.reshape(B, N * D)
    gt_r = gt.reshape(B, N * D)

    nv = (sigma.astype(f32) * sigma.astype(f32)).reshape(1, 1)
    w2 = weights.astype(f32).reshape(B, 1)

    out = pl.pallas_call(
        _vmse_kernel,
        grid=(1,),
        in_specs=[
            pl.BlockSpec(memory_space=pltpu.SMEM),
            pl.BlockSpec((B, N * D), lambda i: (0, 0)),
            pl.BlockSpec((B, N * D), lambda i: (0, 0)),
            pl.BlockSpec((B, 1), lambda i: (0, 0)),
        ],
        out_specs=pl.BlockSpec((B, 128), lambda i: (0, 0)),
        out_shape=jax.ShapeDtypeStruct((B, 128), f32),
        scratch_shapes=[pltpu.VMEM((N, 8, B), f32),
                        pltpu.VMEM((N, 8, B), f32)],
        compiler_params=pltpu.CompilerParams(
            dimension_semantics=("arbitrary",),
            vmem_limit_bytes=48 * 1024 * 1024,
        ),
    )(nv, pred_r, gt_r, w2)
    return out[:, 0]


# speedup vs baseline: 1.0485x; 1.0485x over previous
import jax
import jax.numpy as jnp
from jax.experimental import pallas as pl
from jax.experimental.pallas import tpu as pltpu

_UNROLL = 64
_TREE = 32


def _vmse_kernel(nv_ref, a_ref, b_ref, w_ref, o_ref):
    n_pts = a_ref.shape[0]
    bi = a_ref.shape[2]
    b_tot = b_ref.shape[2]

    def dist(n):
        a = a_ref[pl.ds(n, 1)].reshape(8, bi)
        bb = b_ref[pl.ds(n, 1)].reshape(8, b_tot)
        d2 = jax.lax.dot_general(a, bb, (((0,), (0,)), ((), ())),
                                 preferred_element_type=jnp.float32)
        mb = jnp.maximum(d2.astype(jnp.bfloat16),
                         jnp.bfloat16(1e-30))
        return mb * jax.lax.rsqrt(mb)

    parts = []
    for t in range(n_pts // _TREE):
        s = dist(t * _TREE)
        for u in range(1, _TREE):
            s = s + dist(t * _TREE + u)
        parts.append(s.astype(jnp.float32))
    while len(parts) > 1:
        parts = [parts[i] + parts[i + 1] for i in range(0, len(parts), 2)]
    acc = parts[0]

    nv = nv_ref[0, 0]
    norms = acc * (1.0 / n_pts)
    logits = (norms * norms) * (-0.5 / nv)
    m = jnp.max(logits, axis=1, keepdims=True)
    ex = jnp.exp(logits - m)
    lse = jnp.log(jnp.sum(ex, axis=1, keepdims=True)) + m
    rows = jax.lax.broadcasted_iota(jnp.int32, (bi, b_tot), 0)
    cols = jax.lax.broadcasted_iota(jnp.int32, (bi, b_tot), 1)
    diag = jnp.sum(jnp.where(rows == cols, logits, 0.0), axis=1,
                   keepdims=True)
    loss = (lse - diag) * (2.0 * nv) * w_ref[:, :1]
    o_ref[:, :] = jnp.broadcast_to(loss, (bi, 128))


@jax.jit
def kernel(pred, gt, weights, sigma):
    B, N, D = pred.shape
    f32 = jnp.float32
    pred = pred.astype(f32)
    gt = gt.astype(f32)

    pt = pred.transpose(1, 2, 0)
    gtt = gt.transpose(1, 2, 0)
    p2 = jnp.sum(pt * pt, axis=1, keepdims=True)
    g2 = jnp.sum(gtt * gtt, axis=1, keepdims=True)
    ones = jnp.ones((N, 1, B), f32)
    zeros = jnp.zeros((N, 8 - D - 2, B), f32)
    a_full = jnp.concatenate([pt, p2, ones, zeros], axis=1)
    b_full = jnp.concatenate([-2.0 * gtt, ones, g2, zeros],
                             axis=1)

    nv = (sigma.astype(f32) * sigma.astype(f32)).reshape(1, 1)
    w2 = weights.astype(f32).reshape(B, 1)

    out = pl.pallas_call(
        _vmse_kernel,
        grid=(1,),
        in_specs=[
            pl.BlockSpec(memory_space=pltpu.SMEM),
            pl.BlockSpec((N, 8, B), lambda i: (0, 0, 0)),
            pl.BlockSpec((N, 8, B), lambda i: (0, 0, 0)),
            pl.BlockSpec((B, 1), lambda i: (0, 0)),
        ],
        out_specs=pl.BlockSpec((B, 128), lambda i: (0, 0)),
        out_shape=jax.ShapeDtypeStruct((B, 128), f32),
        compiler_params=pltpu.CompilerParams(
            dimension_semantics=("arbitrary",),
            vmem_limit_bytes=48 * 1024 * 1024,
        ),
    )(nv, a_full, b_full, w2)
    return out[:, 0]
